# Initial kernel scaffold; baseline (speedup 1.0000x reference)
import jax
import jax.numpy as jnp
from jax.experimental import pallas as pl


def kernel(positions, table_0, table_1, table_2, table_3, table_4, table_5, table_6, table_7, table_8, table_9, table_10, table_11, table_12, table_13, table_14, table_15):
    raise NotImplementedError("write your pallas kernel here")


# speedup vs baseline: 42.3448x; 42.3448x over previous
import dataclasses
import functools

import numpy as np
import jax
import jax.numpy as jnp
from jax import lax
from jax.experimental import pallas as pl
from jax.experimental.pallas import tpu as pltpu
from jax.experimental.pallas import tpu_sc as plsc

_NUM_LEVELS = 16
_N = 262144
_HASH_ROWS = 4096
_MASK = np.int32(4095)
_P1 = np.int32(np.uint32(2654435761).view(np.int32))
_P2 = np.int32(805459861)
_RES = [int(np.ceil(16 * 2.0 ** i)) for i in range(_NUM_LEVELS)]

_NW = 32
_PER_W = _N // _NW
_CH = 2048
_NCH = _PER_W // _CH
_LPP = 8


def _lerp(a, b, w, one_minus_w):
    return a * one_minus_w + b * w


def _encode_body(px_hbm, py_hbm, pz_hbm, tc0_hbm, tc1_hbm, out_a_hbm,
                 out_b_hbm, px_v, py_v, pz_v, t0_v, t1_v, o_v):
    outs = (out_a_hbm, out_b_hbm)
    cid = lax.axis_index("c")
    sid = lax.axis_index("s")
    wid = sid * 2 + cid
    base_w = wid * _PER_W
    iot = lax.iota(jnp.int32, 16)

    for p in range(_NUM_LEVELS // _LPP):
        pltpu.sync_copy(tc0_hbm.at[pl.ds(p * _LPP, _LPP)], t0_v)
        pltpu.sync_copy(tc1_hbm.at[pl.ds(p * _LPP, _LPP)], t1_v)
        for ch in range(_NCH):
            base = base_w + ch * _CH
            pltpu.sync_copy(px_hbm.at[pl.ds(base, _CH)], px_v)
            pltpu.sync_copy(py_hbm.at[pl.ds(base, _CH)], py_v)
            pltpu.sync_copy(pz_hbm.at[pl.ds(base, _CH)], pz_v)

            @pl.loop(0, _CH // 16)
            def _(pb):
                off = pb * 16
                x = px_v[pl.ds(off, 16)]
                y = py_v[pl.ds(off, 16)]
                z = pz_v[pl.ds(off, 16)]
                row = iot + off
                for li in range(_LPP):
                    l = p * _LPP + li
                    rf = np.float32(_RES[l] - 1)
                    rm1 = np.int32(_RES[l] - 1)
                    sx = (x + 1.0) * 0.5 * rf
                    sy = (y + 1.0) * 0.5 * rf
                    sz = (z + 1.0) * 0.5 * rf
                    ix = sx.astype(jnp.int32)
                    iy = sy.astype(jnp.int32)
                    iz = sz.astype(jnp.int32)
                    wx = sx - ix.astype(jnp.float32)
                    wy = sy - iy.astype(jnp.float32)
                    wz = sz - iz.astype(jnp.float32)
                    x1 = jnp.minimum(ix + 1, rm1)
                    y1 = jnp.minimum(iy + 1, rm1)
                    z1 = jnp.minimum(iz + 1, rm1)
                    hy0 = iy * _P1
                    hy1 = y1 * _P1
                    hz0 = iz * _P2
                    hz1 = z1 * _P2
                    e00 = ix ^ hy0
                    e01 = ix ^ hy1
                    e10 = x1 ^ hy0
                    e11 = x1 ^ hy1
                    h = [
                        (e00 ^ hz0) & _MASK,
                        (e00 ^ hz1) & _MASK,
                        (e01 ^ hz0) & _MASK,
                        (e01 ^ hz1) & _MASK,
                        (e10 ^ hz0) & _MASK,
                        (e10 ^ hz1) & _MASK,
                        (e11 ^ hz0) & _MASK,
                        (e11 ^ hz1) & _MASK,
                    ]
                    lvl = jnp.full((16,), li, jnp.int32)
                    g0 = [plsc.load_gather(t0_v, [lvl, hj]) for hj in h]
                    g1 = [plsc.load_gather(t1_v, [lvl, hj]) for hj in h]
                    owx = 1.0 - wx
                    owy = 1.0 - wy
                    owz = 1.0 - wz
                    for c, g in ((0, g0), (1, g1)):
                        c00 = _lerp(g[0], g[1], wx, owx)
                        c01 = _lerp(g[2], g[3], wx, owx)
                        c10 = _lerp(g[4], g[5], wx, owx)
                        c11 = _lerp(g[6], g[7], wx, owx)
                        c0 = _lerp(c00, c01, wy, owy)
                        c1 = _lerp(c10, c11, wy, owy)
                        val = _lerp(c0, c1, wz, owz)
                        col = jnp.full((16,), 2 * li + c, jnp.int32)
                        plsc.store_scatter(o_v, [row, col], val)

            pltpu.sync_copy(o_v, outs[p].at[pl.ds(base, _CH)])


@jax.jit
def _sc_encode(px, py, pz, tc0, tc1):
    mesh = plsc.VectorSubcoreMesh(core_axis_name="c", subcore_axis_name="s")
    cp = pltpu.CompilerParams()
    for fld, val in (("needs_layout_passes", False),
                     ("use_tc_tiling_on_sc", False)):
        if fld in pltpu.CompilerParams.__dataclass_fields__:
            cp = dataclasses.replace(cp, **{fld: val})
    f = functools.partial(
        pl.kernel,
        compiler_params=cp,
        out_type=(jax.ShapeDtypeStruct((_N, 2 * _LPP), jnp.float32),
                  jax.ShapeDtypeStruct((_N, 2 * _LPP), jnp.float32)),
        mesh=mesh,
        scratch_types=[
            pltpu.VMEM((_CH,), jnp.float32),
            pltpu.VMEM((_CH,), jnp.float32),
            pltpu.VMEM((_CH,), jnp.float32),
            pltpu.VMEM((_LPP, _HASH_ROWS), jnp.float32),
            pltpu.VMEM((_LPP, _HASH_ROWS), jnp.float32),
            pltpu.VMEM((_CH, 2 * _LPP), jnp.float32),
        ],
    )(_encode_body)
    out_a, out_b = f(px, py, pz, tc0, tc1)
    return jnp.concatenate([out_a, out_b], axis=-1)


def kernel(positions, table_0, table_1, table_2, table_3, table_4, table_5,
           table_6, table_7, table_8, table_9, table_10, table_11, table_12,
           table_13, table_14, table_15):
    tables = [table_0, table_1, table_2, table_3, table_4, table_5, table_6,
              table_7, table_8, table_9, table_10, table_11, table_12,
              table_13, table_14, table_15]
    px = positions[:, 0]
    py = positions[:, 1]
    pz = positions[:, 2]
    tc0 = jnp.stack([t[:_HASH_ROWS, 0] for t in tables])
    tc1 = jnp.stack([t[:_HASH_ROWS, 1] for t in tables])
    return _sc_encode(px, py, pz, tc0, tc1)


# speedup vs baseline: 67.2868x; 1.5890x over previous
import dataclasses
import functools

import numpy as np
import jax
import jax.numpy as jnp
from jax import lax
from jax.experimental import pallas as pl
from jax.experimental.pallas import tpu as pltpu
from jax.experimental.pallas import tpu_sc as plsc

_NUM_LEVELS = 16
_N = 262144
_HASH_ROWS = 4096
_MASK = np.int32(4095)
_P1 = np.int32(np.uint32(2654435761).view(np.int32))
_P2 = np.int32(805459861)
_RES = [int(np.ceil(16 * 2.0 ** i)) for i in range(_NUM_LEVELS)]

_NW = 32
_PER_W = _N // _NW
_CH = 2048
_NCH = _PER_W // _CH
_LPP = 8


def _lerp(a, b, w, one_minus_w):
    return a * one_minus_w + b * w


def _encode_body(px_hbm, py_hbm, pz_hbm, tc0_hbm, tc1_hbm, out_hbm,
                 px_v, py_v, pz_v, t0_v, t1_v, o_v):
    cid = lax.axis_index("c")
    sid = lax.axis_index("s")
    wid = sid * 2 + cid
    base_w = wid * _PER_W

    for p in range(_NUM_LEVELS // _LPP):
        pltpu.sync_copy(tc0_hbm.at[pl.ds(p * _LPP, _LPP)], t0_v)
        pltpu.sync_copy(tc1_hbm.at[pl.ds(p * _LPP, _LPP)], t1_v)
        for ch in range(_NCH):
            base = base_w + ch * _CH
            pltpu.sync_copy(px_hbm.at[pl.ds(base, _CH)], px_v)
            pltpu.sync_copy(py_hbm.at[pl.ds(base, _CH)], py_v)
            pltpu.sync_copy(pz_hbm.at[pl.ds(base, _CH)], pz_v)

            @pl.loop(0, _CH // 16)
            def _(pb):
                off = pb * 16
                x = px_v[pl.ds(off, 16)]
                y = py_v[pl.ds(off, 16)]
                z = pz_v[pl.ds(off, 16)]
                for li in range(_LPP):
                    l = p * _LPP + li
                    rf = np.float32(_RES[l] - 1)
                    rm1 = np.int32(_RES[l] - 1)
                    sx = (x + 1.0) * 0.5 * rf
                    sy = (y + 1.0) * 0.5 * rf
                    sz = (z + 1.0) * 0.5 * rf
                    ix = sx.astype(jnp.int32)
                    iy = sy.astype(jnp.int32)
                    iz = sz.astype(jnp.int32)
                    wx = sx - ix.astype(jnp.float32)
                    wy = sy - iy.astype(jnp.float32)
                    wz = sz - iz.astype(jnp.float32)
                    x1 = jnp.minimum(ix + 1, rm1)
                    y1 = jnp.minimum(iy + 1, rm1)
                    z1 = jnp.minimum(iz + 1, rm1)
                    hy0 = iy * _P1
                    hy1 = y1 * _P1
                    hz0 = iz * _P2
                    hz1 = z1 * _P2
                    e00 = ix ^ hy0
                    e01 = ix ^ hy1
                    e10 = x1 ^ hy0
                    e11 = x1 ^ hy1
                    h = [
                        (e00 ^ hz0) & _MASK,
                        (e00 ^ hz1) & _MASK,
                        (e01 ^ hz0) & _MASK,
                        (e01 ^ hz1) & _MASK,
                        (e10 ^ hz0) & _MASK,
                        (e10 ^ hz1) & _MASK,
                        (e11 ^ hz0) & _MASK,
                        (e11 ^ hz1) & _MASK,
                    ]
                    lvl = jnp.full((16,), li, jnp.int32)
                    g0 = [plsc.load_gather(t0_v, [lvl, hj]) for hj in h]
                    g1 = [plsc.load_gather(t1_v, [lvl, hj]) for hj in h]
                    owx = 1.0 - wx
                    owy = 1.0 - wy
                    owz = 1.0 - wz
                    for c, g in ((0, g0), (1, g1)):
                        c00 = _lerp(g[0], g[1], wx, owx)
                        c01 = _lerp(g[2], g[3], wx, owx)
                        c10 = _lerp(g[4], g[5], wx, owx)
                        c11 = _lerp(g[6], g[7], wx, owx)
                        c0 = _lerp(c00, c01, wy, owy)
                        c1 = _lerp(c10, c11, wy, owy)
                        val = _lerp(c0, c1, wz, owz)
                        o_v[2 * li + c, pl.ds(off, 16)] = val

            pltpu.sync_copy(
                o_v, out_hbm.at[pl.ds(p * 2 * _LPP, 2 * _LPP), pl.ds(base, _CH)])


@jax.jit
def _sc_encode(px, py, pz, tc0, tc1):
    mesh = plsc.VectorSubcoreMesh(core_axis_name="c", subcore_axis_name="s")
    cp = pltpu.CompilerParams()
    for fld, val in (("needs_layout_passes", False),
                     ("use_tc_tiling_on_sc", False)):
        if fld in pltpu.CompilerParams.__dataclass_fields__:
            cp = dataclasses.replace(cp, **{fld: val})
    f = functools.partial(
        pl.kernel,
        compiler_params=cp,
        out_type=jax.ShapeDtypeStruct((2 * _NUM_LEVELS, _N), jnp.float32),
        mesh=mesh,
        scratch_types=[
            pltpu.VMEM((_CH,), jnp.float32),
            pltpu.VMEM((_CH,), jnp.float32),
            pltpu.VMEM((_CH,), jnp.float32),
            pltpu.VMEM((_LPP, _HASH_ROWS), jnp.float32),
            pltpu.VMEM((_LPP, _HASH_ROWS), jnp.float32),
            pltpu.VMEM((2 * _LPP, _CH), jnp.float32),
        ],
    )(_encode_body)
    out_t = f(px, py, pz, tc0, tc1)
    return out_t.T


def kernel(positions, table_0, table_1, table_2, table_3, table_4, table_5,
           table_6, table_7, table_8, table_9, table_10, table_11, table_12,
           table_13, table_14, table_15):
    tables = [table_0, table_1, table_2, table_3, table_4, table_5, table_6,
              table_7, table_8, table_9, table_10, table_11, table_12,
              table_13, table_14, table_15]
    px = positions[:, 0]
    py = positions[:, 1]
    pz = positions[:, 2]
    tc0 = jnp.stack([t[:_HASH_ROWS, 0] for t in tables])
    tc1 = jnp.stack([t[:_HASH_ROWS, 1] for t in tables])
    return _sc_encode(px, py, pz, tc0, tc1)


# speedup vs baseline: 83.9543x; 1.2477x over previous
import dataclasses
import functools

import numpy as np
import jax
import jax.numpy as jnp
from jax import lax
from jax.experimental import pallas as pl
from jax.experimental.pallas import tpu as pltpu
from jax.experimental.pallas import tpu_sc as plsc

_NUM_LEVELS = 16
_N = 262144
_HASH_ROWS = 4096
_MASK = np.int32(4095)
_P1 = np.int32(np.uint32(2654435761).view(np.int32))
_P2 = np.int32(805459861)
_RES = [int(np.ceil(16 * 2.0 ** i)) for i in range(_NUM_LEVELS)]

_NW = 32
_PER_W = _N // _NW
_CH = 1024
_NCH = _PER_W // _CH
_HI16 = np.int32(np.uint32(0xFFFF0000).view(np.int32))


def _lerp(a, b, w, one_minus_w):
    return a * one_minus_w + b * w


def _f32_hi(word):
    return lax.bitcast_convert_type(word & _HI16, jnp.float32)


def _f32_lo(word):
    return lax.bitcast_convert_type(lax.shift_left(word, 16), jnp.float32)


def _encode_body(px_hbm, py_hbm, pz_hbm, tp_hbm, out_hbm,
                 px_v, py_v, pz_v, t_v, o_v):
    cid = lax.axis_index("c")
    sid = lax.axis_index("s")
    wid = sid * 2 + cid
    base_w = wid * _PER_W

    pltpu.sync_copy(tp_hbm, t_v)
    for ch in range(_NCH):
        base = base_w + ch * _CH
        pltpu.sync_copy(px_hbm.at[pl.ds(base, _CH)], px_v)
        pltpu.sync_copy(py_hbm.at[pl.ds(base, _CH)], py_v)
        pltpu.sync_copy(pz_hbm.at[pl.ds(base, _CH)], pz_v)

        @pl.loop(0, _CH // 16)
        def _(pb):
            off = pb * 16
            x = px_v[pl.ds(off, 16)]
            y = py_v[pl.ds(off, 16)]
            z = pz_v[pl.ds(off, 16)]
            for l in range(_NUM_LEVELS):
                rf = np.float32(_RES[l] - 1)
                rm1 = np.int32(_RES[l] - 1)
                sx = (x + 1.0) * 0.5 * rf
                sy = (y + 1.0) * 0.5 * rf
                sz = (z + 1.0) * 0.5 * rf
                ix = sx.astype(jnp.int32)
                iy = sy.astype(jnp.int32)
                iz = sz.astype(jnp.int32)
                wx = sx - ix.astype(jnp.float32)
                wy = sy - iy.astype(jnp.float32)
                wz = sz - iz.astype(jnp.float32)
                x1 = jnp.minimum(ix + 1, rm1)
                y1 = jnp.minimum(iy + 1, rm1)
                z1 = jnp.minimum(iz + 1, rm1)
                hy0 = iy * _P1
                hy1 = y1 * _P1
                hz0 = iz * _P2
                hz1 = z1 * _P2
                e00 = ix ^ hy0
                e01 = ix ^ hy1
                e10 = x1 ^ hy0
                e11 = x1 ^ hy1
                h = [
                    (e00 ^ hz0) & _MASK,
                    (e00 ^ hz1) & _MASK,
                    (e01 ^ hz0) & _MASK,
                    (e01 ^ hz1) & _MASK,
                    (e10 ^ hz0) & _MASK,
                    (e10 ^ hz1) & _MASK,
                    (e11 ^ hz0) & _MASK,
                    (e11 ^ hz1) & _MASK,
                ]
                lvl = jnp.full((16,), l, jnp.int32)
                g = [plsc.bitcast(plsc.load_gather(t_v, [lvl, hj]),
                                  jnp.bfloat16)
                     for hj in h]
                fmt = plsc.PackFormat.INTERLEAVED
                wxp = plsc.pack(wx, wx, format=fmt)
                wyp = plsc.pack(wy, wy, format=fmt)
                wzp = plsc.pack(wz, wz, format=fmt)
                owx = 1.0 - wxp
                owy = 1.0 - wyp
                owz = 1.0 - wzp
                c00 = _lerp(g[0], g[1], wxp, owx)
                c01 = _lerp(g[2], g[3], wxp, owx)
                c10 = _lerp(g[4], g[5], wxp, owx)
                c11 = _lerp(g[6], g[7], wxp, owx)
                c0 = _lerp(c00, c01, wyp, owy)
                c1 = _lerp(c10, c11, wyp, owy)
                val = _lerp(c0, c1, wzp, owz)
                w = plsc.bitcast(val, jnp.int32)
                o_v[2 * l, pl.ds(off, 16)] = _f32_lo(w)
                o_v[2 * l + 1, pl.ds(off, 16)] = _f32_hi(w)

        pltpu.sync_copy(o_v, out_hbm.at[:, pl.ds(base, _CH)])


@jax.jit
def _sc_encode(px, py, pz, tp):
    mesh = plsc.VectorSubcoreMesh(core_axis_name="c", subcore_axis_name="s")
    cp = pltpu.CompilerParams()
    for fld, val in (("needs_layout_passes", False),
                     ("use_tc_tiling_on_sc", False)):
        if fld in pltpu.CompilerParams.__dataclass_fields__:
            cp = dataclasses.replace(cp, **{fld: val})
    f = functools.partial(
        pl.kernel,
        compiler_params=cp,
        out_type=jax.ShapeDtypeStruct((2 * _NUM_LEVELS, _N), jnp.float32),
        mesh=mesh,
        scratch_types=[
            pltpu.VMEM((_CH,), jnp.float32),
            pltpu.VMEM((_CH,), jnp.float32),
            pltpu.VMEM((_CH,), jnp.float32),
            pltpu.VMEM((_NUM_LEVELS, _HASH_ROWS), jnp.int32),
            pltpu.VMEM((2 * _NUM_LEVELS, _CH), jnp.float32),
        ],
    )(_encode_body)
    out_t = f(px, py, pz, tp)
    return out_t.T


def kernel(positions, table_0, table_1, table_2, table_3, table_4, table_5,
           table_6, table_7, table_8, table_9, table_10, table_11, table_12,
           table_13, table_14, table_15):
    tables = [table_0, table_1, table_2, table_3, table_4, table_5, table_6,
              table_7, table_8, table_9, table_10, table_11, table_12,
              table_13, table_14, table_15]
    px = positions[:, 0]
    py = positions[:, 1]
    pz = positions[:, 2]
    packed = []
    for t in tables:
        bits = lax.bitcast_convert_type(
            t[:_HASH_ROWS].astype(jnp.bfloat16), jnp.uint16)
        word = bits[:, 0].astype(jnp.uint32) | (
            bits[:, 1].astype(jnp.uint32) << 16)
        packed.append(lax.bitcast_convert_type(word, jnp.int32))
    tp = jnp.stack(packed)
    return _sc_encode(px, py, pz, tp)
